# Initial kernel scaffold; baseline (speedup 1.0000x reference)
import jax
import jax.numpy as jnp
from jax.experimental import pallas as pl


def kernel(x, mask, edges, W_gcn, b_gcn, W_lin, b_lin):
    raise NotImplementedError("write your pallas kernel here")


# speedup vs baseline: 12.1065x; 12.1065x over previous
import functools

import jax
import jax.numpy as jnp
from jax import lax
from jax.experimental import pallas as pl
from jax.experimental.pallas import tpu as pltpu
from jax.experimental.pallas import tpu_sc as plsc

N, D, E = 10000, 128, 320000
NC, NS = 2, 16
NW = NC * NS
EPW = E // NW
CH = 80
NCH = EPW // CH
NP = 10240
RPT = NP // NS


_mesh = plsc.VectorSubcoreMesh(core_axis_name="c", subcore_axis_name="s")


@functools.partial(
    pl.kernel,
    out_type=jax.ShapeDtypeStruct((NC, NP, D), jnp.float32),
    mesh=_mesh,
    scratch_types=[
        pltpu.VMEM((CH,), jnp.int32),
        pltpu.VMEM((CH, D), jnp.float32),
        pltpu.VMEM_SHARED((NP, D), jnp.float32),
    ],
)
def _deg_sc(col_hbm, ones_hbm, zeros_hbm, out_hbm, idx_v, ones_v, shared_deg):
    c = lax.axis_index("c")
    s = lax.axis_index("s")
    wid = s * NC + c
    pltpu.sync_copy(ones_hbm, ones_v)
    pltpu.sync_copy(zeros_hbm, shared_deg.at[pl.ds(s * RPT, RPT)])
    plsc.subcore_barrier()
    base = wid * EPW

    def body(k, carry):
        off = pl.multiple_of(base + k * CH, 8)
        pltpu.sync_copy(col_hbm.at[pl.ds(off, CH)], idx_v)
        pltpu.sync_copy(ones_v, shared_deg.at[idx_v], add=True)
        return carry

    lax.fori_loop(0, NCH, body, 0)
    plsc.subcore_barrier()
    pltpu.sync_copy(shared_deg.at[pl.ds(s * RPT, RPT)],
                    out_hbm.at[c, pl.ds(s * RPT, RPT)])


@functools.partial(
    pl.kernel,
    out_type=jax.ShapeDtypeStruct((NC, NP, D), jnp.float32),
    mesh=_mesh,
    scratch_types=[
        pltpu.VMEM((CH,), jnp.int32),
        pltpu.VMEM((CH,), jnp.int32),
        pltpu.VMEM((CH, D), jnp.float32),
        pltpu.VMEM_SHARED((NP, D), jnp.float32),
        pltpu.SemaphoreType.DMA,
    ],
)
def _agg_sc(row_hbm, col_hbm, hs_hbm, zeros_hbm, out_hbm,
            ridx_v, cidx_v, rows_v, shared_agg, sem):
    c = lax.axis_index("c")
    s = lax.axis_index("s")
    wid = s * NC + c
    pltpu.sync_copy(zeros_hbm, shared_agg.at[pl.ds(s * RPT, RPT)])
    plsc.subcore_barrier()
    base = wid * EPW

    def body(k, carry):
        off = pl.multiple_of(base + k * CH, 8)
        pltpu.sync_copy(row_hbm.at[pl.ds(off, CH)], ridx_v)
        pltpu.sync_copy(col_hbm.at[pl.ds(off, CH)], cidx_v)
        pltpu.async_copy(hs_hbm.at[ridx_v], rows_v, sem).wait()
        pltpu.sync_copy(rows_v, shared_agg.at[cidx_v], add=True)
        return carry

    lax.fori_loop(0, NCH, body, 0)
    plsc.subcore_barrier()
    pltpu.sync_copy(shared_agg.at[pl.ds(s * RPT, RPT)],
                    out_hbm.at[c, pl.ds(s * RPT, RPT)])


BR = 1024


def _dis_from_degp(degp):
    deg = degp[0, :, 0] + degp[1, :, 0]
    return jnp.where(deg > 0.5, lax.rsqrt(jnp.maximum(deg, 1.0)), 0.0)


def _tc1_body(x_ref, mk_ref, degp_ref, w_ref, hs_ref):
    dis = _dis_from_degp(degp_ref[...])
    nodes = x_ref[...] * mk_ref[...]
    h = jnp.dot(nodes, w_ref[...], preferred_element_type=jnp.float32)
    hs_ref[...] = h * dis[:, None]


def _tc2_body(aggp_ref, degp_ref, x_ref, mk_ref, bg_ref, wl_ref, bl_ref, y_ref):
    dis = _dis_from_degp(degp_ref[...])
    a = (aggp_ref[0] + aggp_ref[1]) * dis[:, None] + bg_ref[...]
    g = a * 0.5 * (1.0 + lax.erf(a * 0.7071067811865476))
    out = jnp.dot(g, wl_ref[...], preferred_element_type=jnp.float32) + bl_ref[...]
    x = x_ref[...]
    y_ref[...] = jnp.where(mk_ref[...] > 0, x + out, x)


_tc1 = pl.pallas_call(
    _tc1_body,
    grid=(NP // BR,),
    in_specs=[
        pl.BlockSpec((BR, D), lambda j: (j, 0)),
        pl.BlockSpec((BR, 1), lambda j: (j, 0)),
        pl.BlockSpec((NC, BR, D), lambda j: (0, j, 0)),
        pl.BlockSpec((D, D), lambda j: (0, 0)),
    ],
    out_specs=pl.BlockSpec((BR, D), lambda j: (j, 0)),
    out_shape=jax.ShapeDtypeStruct((N, D), jnp.float32),
)

_tc2 = pl.pallas_call(
    _tc2_body,
    grid=(NP // BR,),
    in_specs=[
        pl.BlockSpec((NC, BR, D), lambda j: (0, j, 0)),
        pl.BlockSpec((NC, BR, D), lambda j: (0, j, 0)),
        pl.BlockSpec((BR, D), lambda j: (j, 0)),
        pl.BlockSpec((BR, 1), lambda j: (j, 0)),
        pl.BlockSpec((1, D), lambda j: (0, 0)),
        pl.BlockSpec((D, D), lambda j: (0, 0)),
        pl.BlockSpec((1, D), lambda j: (0, 0)),
    ],
    out_specs=pl.BlockSpec((BR, D), lambda j: (j, 0)),
    out_shape=jax.ShapeDtypeStruct((N, D), jnp.float32),
)


def kernel(x, mask, edges, W_gcn, b_gcn, W_lin, b_lin):
    row = edges[0]
    col = edges[1]
    mk = mask.astype(jnp.float32).reshape(N, 1)
    onesD = jnp.ones((CH, D), jnp.float32)
    zerosD = jnp.zeros((RPT, D), jnp.float32)

    degp = _deg_sc(col, onesD, zerosD)
    hs = _tc1(x, mk, degp, W_gcn)
    aggp = _agg_sc(row, col, hs, zerosD)
    return _tc2(aggp, degp, x, mk, b_gcn.reshape(1, D), W_lin, b_lin.reshape(1, D))


# speedup vs baseline: 16.2941x; 1.3459x over previous
import functools

import jax
import jax.numpy as jnp
from jax import lax
from jax.experimental import pallas as pl
from jax.experimental.pallas import tpu as pltpu
from jax.experimental.pallas import tpu_sc as plsc

N, D, E = 10000, 128, 320000
NC, NS = 2, 16
NW = NC * NS
EPW = E // NW
CH = 80
NCH = EPW // CH
NP = 10240
RPT = NP // NS


_mesh = plsc.VectorSubcoreMesh(core_axis_name="c", subcore_axis_name="s")


@functools.partial(
    pl.kernel,
    out_type=jax.ShapeDtypeStruct((NC, NP, D), jnp.float32),
    mesh=_mesh,
    scratch_types=[
        pltpu.VMEM((CH,), jnp.int32),
        pltpu.VMEM((CH, D), jnp.float32),
        pltpu.VMEM_SHARED((NP, D), jnp.float32),
    ],
)
def _deg_sc(col_hbm, ones_hbm, zeros_hbm, out_hbm, idx_v, ones_v, shared_deg):
    c = lax.axis_index("c")
    s = lax.axis_index("s")
    wid = s * NC + c
    pltpu.sync_copy(ones_hbm, ones_v)
    pltpu.sync_copy(zeros_hbm, shared_deg.at[pl.ds(s * RPT, RPT)])
    plsc.subcore_barrier()
    base = wid * EPW

    def body(k, carry):
        off = pl.multiple_of(base + k * CH, 8)
        pltpu.sync_copy(col_hbm.at[pl.ds(off, CH)], idx_v)
        pltpu.sync_copy(ones_v, shared_deg.at[idx_v], add=True)
        return carry

    lax.fori_loop(0, NCH, body, 0)
    plsc.subcore_barrier()
    pltpu.sync_copy(shared_deg.at[pl.ds(s * RPT, RPT)],
                    out_hbm.at[c, pl.ds(s * RPT, RPT)])


@functools.partial(
    pl.kernel,
    out_type=jax.ShapeDtypeStruct((NC, NP, D), jnp.float32),
    mesh=_mesh,
    scratch_types=[
        pltpu.VMEM((CH,), jnp.int32),
        pltpu.VMEM((CH,), jnp.int32),
        pltpu.VMEM((CH,), jnp.int32),
        pltpu.VMEM((CH,), jnp.int32),
        pltpu.VMEM((CH, D), jnp.float32),
        pltpu.VMEM((CH, D), jnp.float32),
        pltpu.VMEM_SHARED((NP, D), jnp.float32),
        pltpu.SemaphoreType.DMA,
        pltpu.SemaphoreType.DMA,
    ],
)
def _agg_sc(row_hbm, col_hbm, hs_hbm, zeros_hbm, out_hbm,
            ridx0, ridx1, cidx0, cidx1, rows0, rows1, shared_agg, sem0, sem1):
    c = lax.axis_index("c")
    s = lax.axis_index("s")
    wid = s * NC + c
    pltpu.sync_copy(zeros_hbm, shared_agg.at[pl.ds(s * RPT, RPT)])
    plsc.subcore_barrier()
    base = wid * EPW

    def off(k):
        return pl.multiple_of(base + k * CH, 8)

    def start(k, ridx, cidx, rows, sem):
        pltpu.sync_copy(row_hbm.at[pl.ds(off(k), CH)], ridx)
        pltpu.sync_copy(col_hbm.at[pl.ds(off(k), CH)], cidx)
        pltpu.async_copy(hs_hbm.at[ridx], rows, sem)

    def drain(rows, sem):
        pltpu.make_async_copy(hs_hbm.at[pl.ds(0, CH)], rows, sem).wait()

    start(0, ridx0, cidx0, rows0, sem0)

    def body(p, carry):
        kA = 2 * p
        start(kA + 1, ridx1, cidx1, rows1, sem1)
        drain(rows0, sem0)
        pltpu.sync_copy(rows0, shared_agg.at[cidx0], add=True)
        start(kA + 2, ridx0, cidx0, rows0, sem0)
        drain(rows1, sem1)
        pltpu.sync_copy(rows1, shared_agg.at[cidx1], add=True)
        return carry

    lax.fori_loop(0, (NCH - 1) // 2, body, 0)
    drain(rows0, sem0)
    pltpu.sync_copy(rows0, shared_agg.at[cidx0], add=True)
    plsc.subcore_barrier()
    pltpu.sync_copy(shared_agg.at[pl.ds(s * RPT, RPT)],
                    out_hbm.at[c, pl.ds(s * RPT, RPT)])


BR = 1024


def _dis_from_degp(degp):
    deg = degp[0, :, 0] + degp[1, :, 0]
    return jnp.where(deg > 0.5, lax.rsqrt(jnp.maximum(deg, 1.0)), 0.0)


def _tch_body(x_ref, mk_ref, w_ref, h_ref):
    nodes = x_ref[...] * mk_ref[...]
    h_ref[...] = jnp.dot(nodes, w_ref[...], preferred_element_type=jnp.float32)


def _tcs_body(h_ref, degp_ref, hs_ref):
    dis = _dis_from_degp(degp_ref[...])
    hs_ref[...] = h_ref[...] * dis[:, None]


def _tc2_body(aggp_ref, degp_ref, x_ref, mk_ref, bg_ref, wl_ref, bl_ref, y_ref):
    dis = _dis_from_degp(degp_ref[...])
    a = (aggp_ref[0] + aggp_ref[1]) * dis[:, None] + bg_ref[...]
    g = a * 0.5 * (1.0 + lax.erf(a * 0.7071067811865476))
    out = jnp.dot(g, wl_ref[...], preferred_element_type=jnp.float32) + bl_ref[...]
    x = x_ref[...]
    y_ref[...] = jnp.where(mk_ref[...] > 0, x + out, x)


_tch = pl.pallas_call(
    _tch_body,
    grid=(NP // BR,),
    in_specs=[
        pl.BlockSpec((BR, D), lambda j: (j, 0)),
        pl.BlockSpec((BR, 1), lambda j: (j, 0)),
        pl.BlockSpec((D, D), lambda j: (0, 0)),
    ],
    out_specs=pl.BlockSpec((BR, D), lambda j: (j, 0)),
    out_shape=jax.ShapeDtypeStruct((N, D), jnp.float32),
)

_tcs = pl.pallas_call(
    _tcs_body,
    grid=(NP // BR,),
    in_specs=[
        pl.BlockSpec((BR, D), lambda j: (j, 0)),
        pl.BlockSpec((NC, BR, D), lambda j: (0, j, 0)),
    ],
    out_specs=pl.BlockSpec((BR, D), lambda j: (j, 0)),
    out_shape=jax.ShapeDtypeStruct((N, D), jnp.float32),
)

_tc2 = pl.pallas_call(
    _tc2_body,
    grid=(NP // BR,),
    in_specs=[
        pl.BlockSpec((NC, BR, D), lambda j: (0, j, 0)),
        pl.BlockSpec((NC, BR, D), lambda j: (0, j, 0)),
        pl.BlockSpec((BR, D), lambda j: (j, 0)),
        pl.BlockSpec((BR, 1), lambda j: (j, 0)),
        pl.BlockSpec((1, D), lambda j: (0, 0)),
        pl.BlockSpec((D, D), lambda j: (0, 0)),
        pl.BlockSpec((1, D), lambda j: (0, 0)),
    ],
    out_specs=pl.BlockSpec((BR, D), lambda j: (j, 0)),
    out_shape=jax.ShapeDtypeStruct((N, D), jnp.float32),
)


def kernel(x, mask, edges, W_gcn, b_gcn, W_lin, b_lin):
    row = edges[0]
    col = edges[1]
    mk = mask.astype(jnp.float32).reshape(N, 1)
    onesD = jnp.ones((CH, D), jnp.float32)
    zerosD = jnp.zeros((RPT, D), jnp.float32)

    degp = _deg_sc(col, onesD, zerosD)
    h = _tch(x, mk, W_gcn)
    hs = _tcs(h, degp)
    aggp = _agg_sc(row, col, hs, zerosD)
    return _tc2(aggp, degp, x, mk, b_gcn.reshape(1, D), W_lin, b_lin.reshape(1, D))


# speedup vs baseline: 21.3967x; 1.3132x over previous
import functools

import jax
import jax.numpy as jnp
from jax import lax
from jax.experimental import pallas as pl
from jax.experimental.pallas import tpu as pltpu
from jax.experimental.pallas import tpu_sc as plsc

N, D, E = 10000, 128, 320000
NC, NS = 2, 16
NW = NC * NS
EPW = E // NW
CH = 80
NCH = EPW // CH
NP = 10240
RPT = NP // NS


_mesh = plsc.VectorSubcoreMesh(core_axis_name="c", subcore_axis_name="s")


@functools.partial(
    pl.kernel,
    out_type=jax.ShapeDtypeStruct((NC, NP, D), jnp.float32),
    mesh=_mesh,
    scratch_types=[
        pltpu.VMEM((CH,), jnp.int32),
        pltpu.VMEM((CH,), jnp.int32),
        pltpu.VMEM((CH, D), jnp.float32),
        pltpu.VMEM_SHARED((NP, D), jnp.float32),
        pltpu.SemaphoreType.DMA,
        pltpu.SemaphoreType.DMA,
    ],
)
def _deg_sc(col_hbm, ones_hbm, zeros_hbm, out_hbm,
            cidx0, cidx1, ones_v, shared_deg, semI0, semI1):
    c = lax.axis_index("c")
    s = lax.axis_index("s")
    wid = s * NC + c
    pltpu.sync_copy(ones_hbm, ones_v)
    pltpu.sync_copy(zeros_hbm, shared_deg.at[pl.ds(s * RPT, RPT)])
    plsc.subcore_barrier()
    base = wid * EPW

    def off(k):
        return pl.multiple_of(base + k * CH, 8)

    def idx_start(k, cidx, semI):
        pltpu.async_copy(col_hbm.at[pl.ds(off(k), CH)], cidx, semI)

    def idx_wait(cidx, semI):
        pltpu.make_async_copy(col_hbm.at[pl.ds(0, CH)], cidx, semI).wait()

    idx_start(0, cidx0, semI0)

    def body(p, carry):
        kA = 2 * p
        idx_start(kA + 1, cidx1, semI1)
        idx_wait(cidx0, semI0)
        pltpu.sync_copy(ones_v, shared_deg.at[cidx0], add=True)
        idx_start(kA + 2, cidx0, semI0)
        idx_wait(cidx1, semI1)
        pltpu.sync_copy(ones_v, shared_deg.at[cidx1], add=True)
        return carry

    lax.fori_loop(0, (NCH - 1) // 2, body, 0)
    idx_wait(cidx0, semI0)
    pltpu.sync_copy(ones_v, shared_deg.at[cidx0], add=True)
    plsc.subcore_barrier()
    pltpu.sync_copy(shared_deg.at[pl.ds(s * RPT, RPT)],
                    out_hbm.at[c, pl.ds(s * RPT, RPT)])


@functools.partial(
    pl.kernel,
    out_type=jax.ShapeDtypeStruct((NC, NP, D), jnp.float32),
    mesh=_mesh,
    scratch_types=[
        pltpu.VMEM((CH,), jnp.int32),
        pltpu.VMEM((CH,), jnp.int32),
        pltpu.VMEM((CH,), jnp.int32),
        pltpu.VMEM((CH,), jnp.int32),
        pltpu.VMEM((CH, D), jnp.float32),
        pltpu.VMEM((CH, D), jnp.float32),
        pltpu.VMEM_SHARED((NP, D), jnp.float32),
        pltpu.SemaphoreType.DMA,
        pltpu.SemaphoreType.DMA,
        pltpu.SemaphoreType.DMA,
        pltpu.SemaphoreType.DMA,
    ],
)
def _agg_sc(row_hbm, col_hbm, hs_hbm, zeros_hbm, out_hbm,
            ridx0, cidx0, ridx1, cidx1, rows0, rows1, shared_agg,
            semI0, semI1, semG0, semG1):
    c = lax.axis_index("c")
    s = lax.axis_index("s")
    wid = s * NC + c
    pltpu.sync_copy(zeros_hbm, shared_agg.at[pl.ds(s * RPT, RPT)])
    plsc.subcore_barrier()
    base = wid * EPW

    def off(k):
        return pl.multiple_of(base + k * CH, 8)

    def idx_start(k, ridx, cidx, semI):
        pltpu.async_copy(row_hbm.at[pl.ds(off(k), CH)], ridx, semI)
        pltpu.async_copy(col_hbm.at[pl.ds(off(k), CH)], cidx, semI)

    def idx_wait(ridx, cidx, semI):
        pltpu.make_async_copy(row_hbm.at[pl.ds(0, CH)], ridx, semI).wait()
        pltpu.make_async_copy(col_hbm.at[pl.ds(0, CH)], cidx, semI).wait()

    def gather_start(ridx, rows, semG):
        pltpu.async_copy(hs_hbm.at[ridx], rows, semG)

    def gather_drain(rows, semG):
        pltpu.make_async_copy(hs_hbm.at[pl.ds(0, CH)], rows, semG).wait()

    idx_start(0, ridx0, cidx0, semI0)
    idx_wait(ridx0, cidx0, semI0)
    gather_start(ridx0, rows0, semG0)
    idx_start(1, ridx1, cidx1, semI1)

    def body(p, carry):
        kA = 2 * p
        idx_wait(ridx1, cidx1, semI1)
        gather_start(ridx1, rows1, semG1)
        gather_drain(rows0, semG0)
        pltpu.sync_copy(rows0, shared_agg.at[cidx0], add=True)
        idx_start(kA + 2, ridx0, cidx0, semI0)
        idx_wait(ridx0, cidx0, semI0)
        gather_start(ridx0, rows0, semG0)
        gather_drain(rows1, semG1)
        pltpu.sync_copy(rows1, shared_agg.at[cidx1], add=True)

        @pl.when(kA + 3 < NCH)
        def _():
            idx_start(kA + 3, ridx1, cidx1, semI1)

        return carry

    lax.fori_loop(0, (NCH - 1) // 2, body, 0)
    gather_drain(rows0, semG0)
    pltpu.sync_copy(rows0, shared_agg.at[cidx0], add=True)
    plsc.subcore_barrier()
    pltpu.sync_copy(shared_agg.at[pl.ds(s * RPT, RPT)],
                    out_hbm.at[c, pl.ds(s * RPT, RPT)])


BR = 1024


def _dis_from_degp(degp):
    deg = degp[0, :, 0] + degp[1, :, 0]
    return jnp.where(deg > 0.5, lax.rsqrt(jnp.maximum(deg, 1.0)), 0.0)


def _tch_body(x_ref, mk_ref, w_ref, h_ref):
    nodes = x_ref[...] * mk_ref[...]
    h_ref[...] = jnp.dot(nodes, w_ref[...], preferred_element_type=jnp.float32)


def _tcs_body(h_ref, degp_ref, hs_ref):
    dis = _dis_from_degp(degp_ref[...])
    hs_ref[...] = h_ref[...] * dis[:, None]


def _tc2_body(aggp_ref, degp_ref, x_ref, mk_ref, bg_ref, wl_ref, bl_ref, y_ref):
    dis = _dis_from_degp(degp_ref[...])
    a = (aggp_ref[0] + aggp_ref[1]) * dis[:, None] + bg_ref[...]
    g = a * 0.5 * (1.0 + lax.erf(a * 0.7071067811865476))
    out = jnp.dot(g, wl_ref[...], preferred_element_type=jnp.float32) + bl_ref[...]
    x = x_ref[...]
    y_ref[...] = jnp.where(mk_ref[...] > 0, x + out, x)


_tch = pl.pallas_call(
    _tch_body,
    grid=(NP // BR,),
    in_specs=[
        pl.BlockSpec((BR, D), lambda j: (j, 0)),
        pl.BlockSpec((BR, 1), lambda j: (j, 0)),
        pl.BlockSpec((D, D), lambda j: (0, 0)),
    ],
    out_specs=pl.BlockSpec((BR, D), lambda j: (j, 0)),
    out_shape=jax.ShapeDtypeStruct((N, D), jnp.float32),
)

_tcs = pl.pallas_call(
    _tcs_body,
    grid=(NP // BR,),
    in_specs=[
        pl.BlockSpec((BR, D), lambda j: (j, 0)),
        pl.BlockSpec((NC, BR, D), lambda j: (0, j, 0)),
    ],
    out_specs=pl.BlockSpec((BR, D), lambda j: (j, 0)),
    out_shape=jax.ShapeDtypeStruct((N, D), jnp.float32),
)

_tc2 = pl.pallas_call(
    _tc2_body,
    grid=(NP // BR,),
    in_specs=[
        pl.BlockSpec((NC, BR, D), lambda j: (0, j, 0)),
        pl.BlockSpec((NC, BR, D), lambda j: (0, j, 0)),
        pl.BlockSpec((BR, D), lambda j: (j, 0)),
        pl.BlockSpec((BR, 1), lambda j: (j, 0)),
        pl.BlockSpec((1, D), lambda j: (0, 0)),
        pl.BlockSpec((D, D), lambda j: (0, 0)),
        pl.BlockSpec((1, D), lambda j: (0, 0)),
    ],
    out_specs=pl.BlockSpec((BR, D), lambda j: (j, 0)),
    out_shape=jax.ShapeDtypeStruct((N, D), jnp.float32),
)


def kernel(x, mask, edges, W_gcn, b_gcn, W_lin, b_lin):
    row = edges[0]
    col = edges[1]
    mk = mask.astype(jnp.float32).reshape(N, 1)
    onesD = jnp.ones((CH, D), jnp.float32)
    zerosD = jnp.zeros((RPT, D), jnp.float32)

    degp = _deg_sc(col, onesD, zerosD)
    h = _tch(x, mk, W_gcn)
    hs = _tcs(h, degp)
    aggp = _agg_sc(row, col, hs, zerosD)
    return _tc2(aggp, degp, x, mk, b_gcn.reshape(1, D), W_lin, b_lin.reshape(1, D))


# speedup vs baseline: 22.1587x; 1.0356x over previous
import functools

import jax
import jax.numpy as jnp
from jax import lax
from jax.experimental import pallas as pl
from jax.experimental.pallas import tpu as pltpu
from jax.experimental.pallas import tpu_sc as plsc

N, D, E = 10000, 128, 320000
NC, NS = 2, 16
NW = NC * NS
EPW = E // NW
CH = 80
NCH = EPW // CH
NP = 10240
RPT = NP // NS


_mesh = plsc.VectorSubcoreMesh(core_axis_name="c", subcore_axis_name="s")


@functools.partial(
    pl.kernel,
    out_type=jax.ShapeDtypeStruct((NC, NP, D), jnp.float32),
    mesh=_mesh,
    scratch_types=[
        pltpu.VMEM((CH,), jnp.int32),
        pltpu.VMEM((CH,), jnp.int32),
        pltpu.VMEM((CH, D), jnp.float32),
        pltpu.VMEM_SHARED((NP, D), jnp.float32),
        pltpu.SemaphoreType.DMA,
        pltpu.SemaphoreType.DMA,
    ],
)
def _deg_sc(col_hbm, ones_hbm, zeros_hbm, out_hbm,
            cidx0, cidx1, ones_v, shared_deg, semI0, semI1):
    c = lax.axis_index("c")
    s = lax.axis_index("s")
    wid = s * NC + c
    pltpu.sync_copy(ones_hbm, ones_v)
    pltpu.sync_copy(zeros_hbm, shared_deg.at[pl.ds(s * RPT, RPT)])
    plsc.subcore_barrier()
    base = wid * EPW

    def off(k):
        return pl.multiple_of(base + k * CH, 8)

    def idx_start(k, cidx, semI):
        pltpu.async_copy(col_hbm.at[pl.ds(off(k), CH)], cidx, semI)

    def idx_wait(cidx, semI):
        pltpu.make_async_copy(col_hbm.at[pl.ds(0, CH)], cidx, semI).wait()

    idx_start(0, cidx0, semI0)

    def body(p, carry):
        kA = 2 * p
        idx_start(kA + 1, cidx1, semI1)
        idx_wait(cidx0, semI0)
        pltpu.sync_copy(ones_v, shared_deg.at[cidx0], add=True)
        idx_start(kA + 2, cidx0, semI0)
        idx_wait(cidx1, semI1)
        pltpu.sync_copy(ones_v, shared_deg.at[cidx1], add=True)
        return carry

    lax.fori_loop(0, (NCH - 1) // 2, body, 0)
    idx_wait(cidx0, semI0)
    pltpu.sync_copy(ones_v, shared_deg.at[cidx0], add=True)
    plsc.subcore_barrier()
    pltpu.sync_copy(shared_deg.at[pl.ds(s * RPT, RPT)],
                    out_hbm.at[c, pl.ds(s * RPT, RPT)])


@functools.partial(
    pl.kernel,
    out_type=jax.ShapeDtypeStruct((NC, NP, D), jnp.float32),
    mesh=_mesh,
    scratch_types=[
        pltpu.VMEM((CH,), jnp.int32),
        pltpu.VMEM((CH,), jnp.int32),
        pltpu.VMEM((CH,), jnp.int32),
        pltpu.VMEM((CH,), jnp.int32),
        pltpu.VMEM((CH,), jnp.int32),
        pltpu.VMEM((CH,), jnp.int32),
        pltpu.VMEM((CH, D), jnp.float32),
        pltpu.VMEM((CH, D), jnp.float32),
        pltpu.VMEM((CH, D), jnp.float32),
        pltpu.VMEM_SHARED((NP, D), jnp.float32),
        pltpu.SemaphoreType.DMA,
        pltpu.SemaphoreType.DMA,
        pltpu.SemaphoreType.DMA,
        pltpu.SemaphoreType.DMA,
        pltpu.SemaphoreType.DMA,
        pltpu.SemaphoreType.DMA,
    ],
)
def _agg_sc(row_hbm, col_hbm, hs_hbm, zeros_hbm, out_hbm,
            ridx0, cidx0, ridx1, cidx1, ridx2, cidx2,
            rows0, rows1, rows2, shared_agg,
            semI0, semI1, semI2, semG0, semG1, semG2):
    c = lax.axis_index("c")
    s = lax.axis_index("s")
    wid = s * NC + c
    pltpu.sync_copy(zeros_hbm, shared_agg.at[pl.ds(s * RPT, RPT)])
    plsc.subcore_barrier()
    base = wid * EPW

    ridx = (ridx0, ridx1, ridx2)
    cidx = (cidx0, cidx1, cidx2)
    rows = (rows0, rows1, rows2)
    semI = (semI0, semI1, semI2)
    semG = (semG0, semG1, semG2)

    def off(k):
        return pl.multiple_of(base + k * CH, 8)

    def idx_start(k, b):
        pltpu.async_copy(row_hbm.at[pl.ds(off(k), CH)], ridx[b], semI[b])
        pltpu.async_copy(col_hbm.at[pl.ds(off(k), CH)], cidx[b], semI[b])

    def idx_wait(b):
        pltpu.make_async_copy(row_hbm.at[pl.ds(0, CH)], ridx[b], semI[b]).wait()
        pltpu.make_async_copy(col_hbm.at[pl.ds(0, CH)], cidx[b], semI[b]).wait()

    def gather_start(b):
        pltpu.async_copy(hs_hbm.at[ridx[b]], rows[b], semG[b])

    def gather_drain(b):
        pltpu.make_async_copy(hs_hbm.at[pl.ds(0, CH)], rows[b], semG[b]).wait()

    for b in range(3):
        idx_start(b, b)
    for b in range(3):
        idx_wait(b)
        gather_start(b)

    def step(k, b):
        gather_drain(b)
        pltpu.sync_copy(rows[b], shared_agg.at[cidx[b]], add=True)

        @pl.when(k + 3 < NCH)
        def _():
            idx_start(k + 3, b)
            idx_wait(b)
            gather_start(b)

    def body(p, carry):
        k = 3 * p
        step(k, 0)
        step(k + 1, 1)
        step(k + 2, 2)
        return carry

    lax.fori_loop(0, NCH // 3, body, 0)
    gather_drain(0)
    pltpu.sync_copy(rows[0], shared_agg.at[cidx[0]], add=True)
    gather_drain(1)
    pltpu.sync_copy(rows[1], shared_agg.at[cidx[1]], add=True)
    plsc.subcore_barrier()
    pltpu.sync_copy(shared_agg.at[pl.ds(s * RPT, RPT)],
                    out_hbm.at[c, pl.ds(s * RPT, RPT)])


BR = 1024


def _dis_from_degp(degp):
    deg = degp[0, :, 0] + degp[1, :, 0]
    return jnp.where(deg > 0.5, lax.rsqrt(jnp.maximum(deg, 1.0)), 0.0)


def _tch_body(x_ref, mk_ref, w_ref, h_ref):
    nodes = x_ref[...] * mk_ref[...]
    h_ref[...] = jnp.dot(nodes, w_ref[...], preferred_element_type=jnp.float32)


def _tcs_body(h_ref, degp_ref, hs_ref):
    dis = _dis_from_degp(degp_ref[...])
    hs_ref[...] = h_ref[...] * dis[:, None]


def _tc2_body(aggp_ref, degp_ref, x_ref, mk_ref, bg_ref, wl_ref, bl_ref, y_ref):
    dis = _dis_from_degp(degp_ref[...])
    a = (aggp_ref[0] + aggp_ref[1]) * dis[:, None] + bg_ref[...]
    g = a * 0.5 * (1.0 + lax.erf(a * 0.7071067811865476))
    out = jnp.dot(g, wl_ref[...], preferred_element_type=jnp.float32) + bl_ref[...]
    x = x_ref[...]
    y_ref[...] = jnp.where(mk_ref[...] > 0, x + out, x)


_tch = pl.pallas_call(
    _tch_body,
    grid=(NP // BR,),
    in_specs=[
        pl.BlockSpec((BR, D), lambda j: (j, 0)),
        pl.BlockSpec((BR, 1), lambda j: (j, 0)),
        pl.BlockSpec((D, D), lambda j: (0, 0)),
    ],
    out_specs=pl.BlockSpec((BR, D), lambda j: (j, 0)),
    out_shape=jax.ShapeDtypeStruct((N, D), jnp.float32),
)

_tcs = pl.pallas_call(
    _tcs_body,
    grid=(NP // BR,),
    in_specs=[
        pl.BlockSpec((BR, D), lambda j: (j, 0)),
        pl.BlockSpec((NC, BR, D), lambda j: (0, j, 0)),
    ],
    out_specs=pl.BlockSpec((BR, D), lambda j: (j, 0)),
    out_shape=jax.ShapeDtypeStruct((N, D), jnp.float32),
)

_tc2 = pl.pallas_call(
    _tc2_body,
    grid=(NP // BR,),
    in_specs=[
        pl.BlockSpec((NC, BR, D), lambda j: (0, j, 0)),
        pl.BlockSpec((NC, BR, D), lambda j: (0, j, 0)),
        pl.BlockSpec((BR, D), lambda j: (j, 0)),
        pl.BlockSpec((BR, 1), lambda j: (j, 0)),
        pl.BlockSpec((1, D), lambda j: (0, 0)),
        pl.BlockSpec((D, D), lambda j: (0, 0)),
        pl.BlockSpec((1, D), lambda j: (0, 0)),
    ],
    out_specs=pl.BlockSpec((BR, D), lambda j: (j, 0)),
    out_shape=jax.ShapeDtypeStruct((N, D), jnp.float32),
)


def kernel(x, mask, edges, W_gcn, b_gcn, W_lin, b_lin):
    row = edges[0]
    col = edges[1]
    mk = mask.astype(jnp.float32).reshape(N, 1)
    onesD = jnp.ones((CH, D), jnp.float32)
    zerosD = jnp.zeros((RPT, D), jnp.float32)

    degp = _deg_sc(col, onesD, zerosD)
    h = _tch(x, mk, W_gcn)
    hs = _tcs(h, degp)
    aggp = _agg_sc(row, col, hs, zerosD)
    return _tc2(aggp, degp, x, mk, b_gcn.reshape(1, D), W_lin, b_lin.reshape(1, D))
